# baseline (speedup 1.0000x reference)
import jax
import jax.numpy as jnp
from jax.experimental import pallas as pl
from jax.experimental.pallas import tpu as pltpu

_C = 512
_R = 256
_RIN = 2048
_RP = 1024
_K = 16
_G = 32


def _elu(v):
    vn = jnp.minimum(v, 0.0)
    return jnp.where(v > 0, v, jnp.tanh(0.5 * vn) * (jnp.exp(vn) + 1.0))


def _in_mlp_kernel(x_ref, dn_ref, w0, b0, w1, b1, w2, b2, o_ref):
    h = x_ref[...] * dn_ref[...]
    h = _elu(jnp.dot(h, w0[...], preferred_element_type=jnp.float32) + b0[...])
    h = _elu(jnp.dot(h, w1[...], preferred_element_type=jnp.float32) + b1[...])
    h = _elu(jnp.dot(h, w2[...], preferred_element_type=jnp.float32) + b2[...])
    o_ref[...] = h


def _ec_kernel(loal_ref, nt_ref, hp_ref, browt_ref, sts_ref, ens_ref,
               w0, b0, w1, b1, o_ref):
    R, C, K = _R, _C, _K
    H = hp_ref.shape[1]
    pid = pl.program_id(0)
    r0 = pid * R
    xi = hp_ref[pl.ds(r0, R), :]
    brow = browt_ref[...]
    ridx = r0 + jax.lax.broadcasted_iota(jnp.int32, (R, 1), 0)
    g32 = jax.lax.broadcasted_iota(jnp.int32, (1, _G), 1)
    oh_g = (brow == g32).astype(jnp.float32)
    lo_row = jax.lax.dot_general(
        oh_g, sts_ref[...], (((1,), (1,)), ((), ())),
        preferred_element_type=jnp.float32).astype(jnp.int32)
    hi_row = jax.lax.dot_general(
        oh_g, ens_ref[...], (((1,), (1,)), ((), ())),
        preferred_element_type=jnp.float32).astype(jnp.int32)
    sqi = jnp.sum(xi * xi, axis=1, keepdims=True)
    lo_al = loal_ref[pid]
    n_t = nt_ref[pid]
    INF = jnp.float32(jnp.inf)
    BIG = jnp.int32(2 ** 30)

    def tile_body(t, carry):
        topv, topi = carry
        base = lo_al + t * C
        cols = hp_ref[pl.ds(base, C), :]
        sqc = jnp.sum(cols * cols, axis=1)[None, :]
        dot = jax.lax.dot_general(
            xi, cols, (((1,), (1,)), ((), ())),
            preferred_element_type=jnp.float32)
        d = sqi + sqc - 2.0 * dot
        cidx = base + jax.lax.broadcasted_iota(jnp.int32, (1, C), 1)
        valid = (cidx >= lo_row) & (cidx < hi_row) & (cidx != ridx)
        d = jnp.where(valid, d, INF)
        for _ in range(K):
            m = jnp.min(d, axis=1, keepdims=True)
            hit = d == m
            im = jnp.min(jnp.where(hit, jnp.broadcast_to(cidx, d.shape), BIG),
                         axis=1, keepdims=True)
            d = jnp.where(hit & (cidx == im), INF, d)
            cm = jnp.max(topv, axis=1, keepdims=True)
            evi = jnp.max(jnp.where(topv == cm, topi, -BIG), axis=1, keepdims=True)
            repl = (topv == cm) & (topi == evi) & (m < cm)
            topv = jnp.where(repl, jnp.broadcast_to(m, topv.shape), topv)
            topi = jnp.where(repl, jnp.broadcast_to(im, topi.shape), topi)
        return topv, topi

    topv0 = jnp.full((R, K), INF)
    topi0 = -1 - jax.lax.broadcasted_iota(jnp.int32, (R, K), 1)
    _, topi = jax.lax.fori_loop(0, n_t, tile_body, (topv0, topi0))

    def gather_body(t, xjs):
        base = lo_al + t * C
        cols = hp_ref[pl.ds(base, C), :]
        cit = jax.lax.broadcasted_iota(jnp.int32, (1, C), 1) + base
        return tuple(
            xjs[k] + jnp.dot((topi[:, k:k + 1] == cit).astype(jnp.float32),
                             cols, preferred_element_type=jnp.float32)
            for k in range(K))

    xjs = jax.lax.fori_loop(
        0, n_t, gather_body,
        tuple(jnp.zeros((R, H), jnp.float32) for _ in range(K)))

    acc = jnp.zeros((R, H), jnp.float32)
    for k in range(K):
        cat = jnp.concatenate([xi, xjs[k] - xi], axis=1)
        m0 = _elu(jnp.dot(cat, w0[...], preferred_element_type=jnp.float32)
                  + b0[...])
        acc = acc + _elu(jnp.dot(m0, w1[...],
                                 preferred_element_type=jnp.float32) + b1[...])
    o_ref[...] = acc


def _out_kernel(h_ref, brow_ref, w0, b0, w1, b1, w2, b2, o_ref, acc_ref):
    pid = pl.program_id(0)
    nlast = pl.num_programs(0) - 1

    @pl.when(pid == 0)
    def _():
        acc_ref[...] = jnp.full(acc_ref.shape, -jnp.inf, jnp.float32)

    hb = h_ref[...]
    brow = brow_ref[...]
    NEG = jnp.float32(-jnp.inf)
    gi = jax.lax.broadcasted_iota(jnp.int32, (_G, 1), 0)

    def seg_body(g, acc):
        val = jnp.max(jnp.where(brow == g, hb, NEG), axis=0, keepdims=True)
        return jnp.where(gi == g, jnp.maximum(acc, val), acc)

    acc_ref[...] = jax.lax.fori_loop(brow[0, 0], brow[_RP - 1, 0] + 1,
                                     seg_body, acc_ref[...])

    @pl.when(pid == nlast)
    def _():
        p = acc_ref[...]
        g = _elu(jnp.dot(p, w0[...], preferred_element_type=jnp.float32) + b0[...])
        g = _elu(jnp.dot(g, w1[...], preferred_element_type=jnp.float32) + b1[...])
        lg = jnp.dot(g, w2[...], preferred_element_type=jnp.float32) + b2[...]
        mx = jnp.max(lg, axis=1, keepdims=True)
        lse = jnp.log(jnp.sum(jnp.exp(lg - mx), axis=1, keepdims=True)) + mx
        o_ref[...] = lg - lse


def kernel(x, batch, datanorm, in_W0, in_b0, in_W1, in_b1, in_W2, in_b2,
           ec0_W0, ec0_b0, ec0_W1, ec0_b1, ec1_W0, ec1_b0, ec1_W1, ec1_b1,
           out_W0, out_b0, out_W1, out_b1, out_W2, out_b2):
    N, DF = x.shape
    H = in_W0.shape[1]
    NC = out_W2.shape[1]
    f32 = jnp.float32
    b = lambda v: v.reshape(1, -1)

    h0 = pl.pallas_call(
        _in_mlp_kernel,
        grid=(N // _RIN,),
        in_specs=[pl.BlockSpec((_RIN, DF), lambda i: (i, 0)),
                  pl.BlockSpec((1, DF), lambda i: (0, 0)),
                  pl.BlockSpec((DF, H), lambda i: (0, 0)),
                  pl.BlockSpec((1, H), lambda i: (0, 0)),
                  pl.BlockSpec((H, H), lambda i: (0, 0)),
                  pl.BlockSpec((1, H), lambda i: (0, 0)),
                  pl.BlockSpec((H, H), lambda i: (0, 0)),
                  pl.BlockSpec((1, H), lambda i: (0, 0))],
        out_specs=pl.BlockSpec((_RIN, H), lambda i: (i, 0)),
        out_shape=jax.ShapeDtypeStruct((N, H), f32),
    )(x, datanorm.reshape(1, DF), in_W0, b(in_b0), in_W1, b(in_b1),
      in_W2, b(in_b2))

    gidx = jnp.arange(_G, dtype=jnp.int32)
    starts = jnp.searchsorted(batch, gidx, side='left').astype(jnp.int32)
    ends = jnp.searchsorted(batch, gidx, side='right').astype(jnp.int32)
    stsf = starts.astype(f32).reshape(1, _G)
    ensf = ends.astype(f32).reshape(1, _G)
    nchunk = N // _R
    lo = starts[batch[::_R]]
    hi = ends[batch[_R - 1::_R]]
    lo_al = (lo // 8) * 8
    n_t = (hi - lo_al + _C - 1) // _C
    batch_col = batch.reshape(N, 1)

    def edge_conv(h, w0, b0v, w1, b1v):
        hp = jnp.concatenate([h, jnp.zeros((_C, H), f32)], axis=0)
        return pl.pallas_call(
            _ec_kernel,
            grid_spec=pltpu.PrefetchScalarGridSpec(
                num_scalar_prefetch=2,
                grid=(nchunk,),
                in_specs=[pl.BlockSpec((N + _C, H), lambda i, *_: (0, 0)),
                          pl.BlockSpec((_R, 1), lambda i, *_: (i, 0)),
                          pl.BlockSpec((1, _G), lambda i, *_: (0, 0)),
                          pl.BlockSpec((1, _G), lambda i, *_: (0, 0)),
                          pl.BlockSpec((2 * H, H), lambda i, *_: (0, 0)),
                          pl.BlockSpec((1, H), lambda i, *_: (0, 0)),
                          pl.BlockSpec((H, H), lambda i, *_: (0, 0)),
                          pl.BlockSpec((1, H), lambda i, *_: (0, 0))],
                out_specs=pl.BlockSpec((_R, H), lambda i, *_: (i, 0)),
            ),
            out_shape=jax.ShapeDtypeStruct((N, H), f32),
        )(lo_al, n_t, hp, batch_col, stsf, ensf, w0, b(b0v), w1, b(b1v))

    h1 = edge_conv(h0, ec0_W0, ec0_b0, ec0_W1, ec0_b1)
    h2 = edge_conv(h1, ec1_W0, ec1_b0, ec1_W1, ec1_b1)

    return pl.pallas_call(
        _out_kernel,
        grid=(N // _RP,),
        in_specs=[pl.BlockSpec((_RP, H), lambda i: (i, 0)),
                  pl.BlockSpec((_RP, 1), lambda i: (i, 0)),
                  pl.BlockSpec((H, H), lambda i: (0, 0)),
                  pl.BlockSpec((1, H), lambda i: (0, 0)),
                  pl.BlockSpec((H, H), lambda i: (0, 0)),
                  pl.BlockSpec((1, H), lambda i: (0, 0)),
                  pl.BlockSpec((H, NC), lambda i: (0, 0)),
                  pl.BlockSpec((1, NC), lambda i: (0, 0))],
        out_specs=pl.BlockSpec((_G, NC), lambda i: (0, 0)),
        out_shape=jax.ShapeDtypeStruct((_G, NC), f32),
        scratch_shapes=[pltpu.VMEM((_G, H), f32)],
    )(h2, batch.reshape(N, 1), out_W0, b(out_b0), out_W1, b(out_b1),
      out_W2, b(out_b2))


# speedup vs baseline: 1.0765x; 1.0765x over previous
import jax
import jax.numpy as jnp
from jax.experimental import pallas as pl
from jax.experimental.pallas import tpu as pltpu

_C = 768
_R = 256
_RIN = 2048
_RP = 1024
_K = 16
_G = 32


def _elu(v):
    vn = jnp.minimum(v, 0.0)
    return jnp.where(v > 0, v, jnp.tanh(0.5 * vn) * (jnp.exp(vn) + 1.0))


def _in_mlp_kernel(x_ref, dn_ref, w0, b0, w1, b1, w2, b2, o_ref):
    h = x_ref[...] * dn_ref[...]
    h = _elu(jnp.dot(h, w0[...], preferred_element_type=jnp.float32) + b0[...])
    h = _elu(jnp.dot(h, w1[...], preferred_element_type=jnp.float32) + b1[...])
    h = _elu(jnp.dot(h, w2[...], preferred_element_type=jnp.float32) + b2[...])
    o_ref[...] = h


def _ec_kernel(loal_ref, nt_ref, hp_ref, brow_ref, sts_ref, ens_ref,
               w0, b0, w1, b1, o_ref):
    R, C, K = _R, _C, _K
    H = hp_ref.shape[1]
    pid = pl.program_id(0)
    r0 = pid * R
    xi = hp_ref[pl.ds(r0, R), :]
    brow = brow_ref[...]
    ridx = r0 + jax.lax.broadcasted_iota(jnp.int32, (R, 1), 0)
    g32 = jax.lax.broadcasted_iota(jnp.int32, (1, _G), 1)
    oh_g = (brow == g32).astype(jnp.float32)
    lo_row = jax.lax.dot_general(
        oh_g, sts_ref[...], (((1,), (1,)), ((), ())),
        preferred_element_type=jnp.float32).astype(jnp.int32)
    hi_row = jax.lax.dot_general(
        oh_g, ens_ref[...], (((1,), (1,)), ((), ())),
        preferred_element_type=jnp.float32).astype(jnp.int32)
    sqi = jnp.sum(xi * xi, axis=1, keepdims=True)
    lo_al = loal_ref[pid]
    n_t = nt_ref[pid]
    INF = jnp.float32(jnp.inf)
    BIG = jnp.int32(2 ** 30)

    def tile_body(t, carry):
        topv, topi = carry
        base = lo_al + t * C
        cols = hp_ref[pl.ds(base, C), :]
        sqc = jnp.sum(cols * cols, axis=1)[None, :]
        dot = jax.lax.dot_general(
            xi, cols, (((1,), (1,)), ((), ())),
            preferred_element_type=jnp.float32)
        d = sqi + sqc - 2.0 * dot
        cidx = base + jax.lax.broadcasted_iota(jnp.int32, (1, C), 1)
        valid = (cidx >= lo_row) & (cidx < hi_row) & (cidx != ridx)
        d = jnp.where(valid, d, INF)
        for _ in range(K):
            m = jnp.min(d, axis=1, keepdims=True)
            hit = d == m
            im = jnp.min(jnp.where(hit, jnp.broadcast_to(cidx, d.shape), BIG),
                         axis=1, keepdims=True)
            d = jnp.where(hit & (cidx == im), INF, d)
            cm = jnp.max(topv, axis=1, keepdims=True)
            evi = jnp.max(jnp.where(topv == cm, topi, -BIG), axis=1, keepdims=True)
            repl = (topv == cm) & (topi == evi) & (m < cm)
            topv = jnp.where(repl, jnp.broadcast_to(m, topv.shape), topv)
            topi = jnp.where(repl, jnp.broadcast_to(im, topi.shape), topi)
        return topv, topi

    topv0 = jnp.full((R, K), INF)
    topi0 = -1 - jax.lax.broadcasted_iota(jnp.int32, (R, K), 1)
    _, topi = jax.lax.fori_loop(0, n_t, tile_body, (topv0, topi0))

    def gather_body(t, xjs):
        base = lo_al + t * C
        cols = hp_ref[pl.ds(base, C), :]
        cit = jax.lax.broadcasted_iota(jnp.int32, (1, C), 1) + base
        return tuple(
            xjs[k] + jnp.dot((topi[:, k:k + 1] == cit).astype(jnp.float32),
                             cols, preferred_element_type=jnp.float32)
            for k in range(K))

    xjs = jax.lax.fori_loop(
        0, n_t, gather_body,
        tuple(jnp.zeros((R, H), jnp.float32) for _ in range(K)))

    acc = jnp.zeros((R, H), jnp.float32)
    for k in range(K):
        cat = jnp.concatenate([xi, xjs[k] - xi], axis=1)
        m0 = _elu(jnp.dot(cat, w0[...], preferred_element_type=jnp.float32)
                  + b0[...])
        acc = acc + _elu(jnp.dot(m0, w1[...],
                                 preferred_element_type=jnp.float32) + b1[...])
    o_ref[...] = acc


def _out_kernel(h_ref, brow_ref, w0, b0, w1, b1, w2, b2, o_ref, acc_ref):
    pid = pl.program_id(0)
    nlast = pl.num_programs(0) - 1

    @pl.when(pid == 0)
    def _():
        acc_ref[...] = jnp.full(acc_ref.shape, -jnp.inf, jnp.float32)

    hb = h_ref[...]
    brow = brow_ref[...]
    NEG = jnp.float32(-jnp.inf)
    gi = jax.lax.broadcasted_iota(jnp.int32, (_G, 1), 0)

    def seg_body(g, acc):
        val = jnp.max(jnp.where(brow == g, hb, NEG), axis=0, keepdims=True)
        return jnp.where(gi == g, jnp.maximum(acc, val), acc)

    acc_ref[...] = jax.lax.fori_loop(brow[0, 0], brow[_RP - 1, 0] + 1,
                                     seg_body, acc_ref[...])

    @pl.when(pid == nlast)
    def _():
        p = acc_ref[...]
        g = _elu(jnp.dot(p, w0[...], preferred_element_type=jnp.float32) + b0[...])
        g = _elu(jnp.dot(g, w1[...], preferred_element_type=jnp.float32) + b1[...])
        lg = jnp.dot(g, w2[...], preferred_element_type=jnp.float32) + b2[...]
        mx = jnp.max(lg, axis=1, keepdims=True)
        lse = jnp.log(jnp.sum(jnp.exp(lg - mx), axis=1, keepdims=True)) + mx
        o_ref[...] = lg - lse


def kernel(x, batch, datanorm, in_W0, in_b0, in_W1, in_b1, in_W2, in_b2,
           ec0_W0, ec0_b0, ec0_W1, ec0_b1, ec1_W0, ec1_b0, ec1_W1, ec1_b1,
           out_W0, out_b0, out_W1, out_b1, out_W2, out_b2):
    N, DF = x.shape
    H = in_W0.shape[1]
    NC = out_W2.shape[1]
    f32 = jnp.float32
    b = lambda v: v.reshape(1, -1)

    h0 = pl.pallas_call(
        _in_mlp_kernel,
        grid=(N // _RIN,),
        in_specs=[pl.BlockSpec((_RIN, DF), lambda i: (i, 0)),
                  pl.BlockSpec((1, DF), lambda i: (0, 0)),
                  pl.BlockSpec((DF, H), lambda i: (0, 0)),
                  pl.BlockSpec((1, H), lambda i: (0, 0)),
                  pl.BlockSpec((H, H), lambda i: (0, 0)),
                  pl.BlockSpec((1, H), lambda i: (0, 0)),
                  pl.BlockSpec((H, H), lambda i: (0, 0)),
                  pl.BlockSpec((1, H), lambda i: (0, 0))],
        out_specs=pl.BlockSpec((_RIN, H), lambda i: (i, 0)),
        out_shape=jax.ShapeDtypeStruct((N, H), f32),
    )(x, datanorm.reshape(1, DF), in_W0, b(in_b0), in_W1, b(in_b1),
      in_W2, b(in_b2))

    gidx = jnp.arange(_G, dtype=jnp.int32)
    starts = jnp.searchsorted(batch, gidx, side='left').astype(jnp.int32)
    ends = jnp.searchsorted(batch, gidx, side='right').astype(jnp.int32)
    stsf = starts.astype(f32).reshape(1, _G)
    ensf = ends.astype(f32).reshape(1, _G)
    nchunk = N // _R
    lo = starts[batch[::_R]]
    hi = ends[batch[_R - 1::_R]]
    lo_al = (lo // 8) * 8
    n_t = (hi - lo_al + _C - 1) // _C
    batch_col = batch.reshape(N, 1)

    def edge_conv(h, w0, b0v, w1, b1v):
        hp = jnp.concatenate([h, jnp.zeros((_C, H), f32)], axis=0)
        return pl.pallas_call(
            _ec_kernel,
            grid_spec=pltpu.PrefetchScalarGridSpec(
                num_scalar_prefetch=2,
                grid=(nchunk,),
                in_specs=[pl.BlockSpec((N + _C, H), lambda i, *_: (0, 0)),
                          pl.BlockSpec((_R, 1), lambda i, *_: (i, 0)),
                          pl.BlockSpec((1, _G), lambda i, *_: (0, 0)),
                          pl.BlockSpec((1, _G), lambda i, *_: (0, 0)),
                          pl.BlockSpec((2 * H, H), lambda i, *_: (0, 0)),
                          pl.BlockSpec((1, H), lambda i, *_: (0, 0)),
                          pl.BlockSpec((H, H), lambda i, *_: (0, 0)),
                          pl.BlockSpec((1, H), lambda i, *_: (0, 0))],
                out_specs=pl.BlockSpec((_R, H), lambda i, *_: (i, 0)),
            ),
            out_shape=jax.ShapeDtypeStruct((N, H), f32),
        )(lo_al, n_t, hp, batch_col, stsf, ensf, w0, b(b0v), w1, b(b1v))

    h1 = edge_conv(h0, ec0_W0, ec0_b0, ec0_W1, ec0_b1)
    h2 = edge_conv(h1, ec1_W0, ec1_b0, ec1_W1, ec1_b1)

    return pl.pallas_call(
        _out_kernel,
        grid=(N // _RP,),
        in_specs=[pl.BlockSpec((_RP, H), lambda i: (i, 0)),
                  pl.BlockSpec((_RP, 1), lambda i: (i, 0)),
                  pl.BlockSpec((H, H), lambda i: (0, 0)),
                  pl.BlockSpec((1, H), lambda i: (0, 0)),
                  pl.BlockSpec((H, H), lambda i: (0, 0)),
                  pl.BlockSpec((1, H), lambda i: (0, 0)),
                  pl.BlockSpec((H, NC), lambda i: (0, 0)),
                  pl.BlockSpec((1, NC), lambda i: (0, 0))],
        out_specs=pl.BlockSpec((_G, NC), lambda i: (0, 0)),
        out_shape=jax.ShapeDtypeStruct((_G, NC), f32),
        scratch_shapes=[pltpu.VMEM((_G, H), f32)],
    )(h2, batch.reshape(N, 1), out_W0, b(out_b0), out_W1, b(out_b1),
      out_W2, b(out_b2))
